# Initial kernel scaffold; baseline (speedup 1.0000x reference)
import functools

import jax
import jax.numpy as jnp
import numpy as np
from jax.experimental import pallas as pl

_NW = 16
_HIDDEN = 4
_NEURONS = 128
_OVERLAP = 0.25
_SIGMA = 0.02
_DOM = (0.0, 1.0)
_N_PTS = 131072

_TILE = 2048


def _geometry():
    w = (_DOM[1] - _DOM[0]) / _NW
    sd = np.zeros((_NW, 2), dtype=np.float32)
    for i in range(_NW):
        sd[i, 0] = _DOM[0] if i == 0 else _DOM[0] + (i - _OVERLAP / 2) * w
        sd[i, 1] = _DOM[1] if i == _NW - 1 else _DOM[0] + (i + 1 + _OVERLAP / 2) * w
    mo = np.zeros(_NW + 1, dtype=np.float32)
    mo[0] = sd[0, 0]
    mo[_NW] = sd[_NW - 1, 1]
    for i in range(1, _NW):
        mo[i] = (sd[i - 1, 1] + sd[i, 0]) / 2
    means = (sd[:, 1] + sd[:, 0]) / 2
    std = (sd[:, 1] - sd[:, 0]) / 2
    return mo, means, std


_MO, _MEANS, _STD = _geometry()


def _fbpinn_kernel(x_ref, w_in_ref, b_in_ref, w_hid_ref, b_hid_ref,
                   w_out_ref, b_out_ref, out_ref):
    x = x_ref[0, :]
    acc = jnp.zeros_like(x)
    for i in range(_NW):
        xl = (x - _MO[i]) * (1.0 / _SIGMA)
        xr = (x - _MO[i + 1]) * (1.0 / _SIGMA)
        window = jax.nn.sigmoid(xl) * jax.nn.sigmoid(-xr)
        mask = window > 0.001
        xn = (x - _MEANS[i]) * (1.0 / _STD[i])
        h = jnp.tanh(xn[:, None] * w_in_ref[i, 0, :][None, :] +
                     b_in_ref[i, :][None, :])
        for j in range(_HIDDEN - 1):
            h = jnp.tanh(
                jax.lax.dot_general(
                    h, w_hid_ref[i, j, :, :],
                    (((1,), (0,)), ((), ())),
                    preferred_element_type=jnp.float32,
                ) + b_hid_ref[i, j, :][None, :])
        out = jnp.sum(h * w_out_ref[i, :, 0][None, :], axis=1) + b_out_ref[i, 0]
        acc = acc + jnp.where(mask, window * out, 0.0)
    out_ref[0, :] = jnp.tanh(x) * acc


@jax.jit
def _fbpinn(x, W_in, b_in, W_hid, b_hid, W_out, b_out):
    n = x.shape[0]
    grid = n // _TILE
    x2 = x.reshape(grid, _TILE)
    full = lambda *s: pl.BlockSpec(s, lambda i: (0,) * len(s))
    pred = pl.pallas_call(
        _fbpinn_kernel,
        grid=(grid,),
        in_specs=[
            pl.BlockSpec((1, _TILE), lambda i: (i, 0)),
            full(_NW, 1, _NEURONS),
            full(_NW, _NEURONS),
            full(_NW, _HIDDEN - 1, _NEURONS, _NEURONS),
            full(_NW, _HIDDEN - 1, _NEURONS),
            full(_NW, _NEURONS, 1),
            full(_NW, 1),
        ],
        out_specs=pl.BlockSpec((1, _TILE), lambda i: (i, 0)),
        out_shape=jax.ShapeDtypeStruct((grid, _TILE), jnp.float32),
    )(x2, W_in, b_in, W_hid, b_hid, W_out, b_out)
    return pred.reshape(n)


def kernel(input, W_in, b_in, W_hid, b_hid, W_out, b_out):
    pred = _fbpinn(input, W_in, b_in, W_hid, b_hid, W_out, b_out)
    flops = _NW * _N_PTS * (2 * _NEURONS +
                            (_HIDDEN - 1) * 2 * _NEURONS * _NEURONS +
                            2 * _NEURONS)
    return (pred, flops)


# speedup vs baseline: 208.1782x; 208.1782x over previous
import jax
import jax.numpy as jnp
import numpy as np
from jax.experimental import pallas as pl

jax.config.update("jax_enable_x64", True)

_NW = 16
_HIDDEN = 4
_NEURONS = 128
_OVERLAP = 0.25
_SIGMA = 0.02
_DOM = (0.0, 1.0)
_N_PTS = 131072

_TILE = 2048


def _geometry():
    w = (_DOM[1] - _DOM[0]) / _NW
    sd = np.zeros((_NW, 2), dtype=np.float32)
    for i in range(_NW):
        sd[i, 0] = _DOM[0] if i == 0 else _DOM[0] + (i - _OVERLAP / 2) * w
        sd[i, 1] = _DOM[1] if i == _NW - 1 else _DOM[0] + (i + 1 + _OVERLAP / 2) * w
    mo = np.zeros(_NW + 1, dtype=np.float32)
    mo[0] = sd[0, 0]
    mo[_NW] = sd[_NW - 1, 1]
    for i in range(1, _NW):
        mo[i] = (sd[i - 1, 1] + sd[i, 0]) / 2
    means = (sd[:, 1] + sd[:, 0]) / 2
    std = (sd[:, 1] - sd[:, 0]) / 2
    return mo, means, std


_MO, _MEANS, _STD = _geometry()


def _fbpinn_kernel(x_ref, w1_ref, b1_ref, w_hid_ref, b_hid_ref,
                   w_out_ref, b_out_ref, out_ref):
    x = x_ref[...]
    sig = [jax.nn.sigmoid((x - float(_MO[k])) * (1.0 / _SIGMA))
           for k in range(_NW + 1)]
    acc = jnp.zeros_like(x)
    for i in range(_NW):
        window = sig[i] * (1.0 - sig[i + 1])
        h = jnp.tanh(w1_ref[:, i:i + 1] * x + b1_ref[:, i:i + 1])
        for j in range(_HIDDEN - 1):
            base = (i * (_HIDDEN - 1) + j) * _NEURONS
            h = jnp.tanh(
                jax.lax.dot_general(
                    w_hid_ref[base:base + _NEURONS, :], h,
                    (((0,), (0,)), ((), ())),
                    preferred_element_type=jnp.float32,
                    precision=jax.lax.Precision.HIGHEST,
                ) + b_hid_ref[:, 3 * i + j:3 * i + j + 1])
        out = (jnp.sum(h * w_out_ref[:, i:i + 1], axis=0, keepdims=True)
               + b_out_ref[:, i:i + 1])
        acc = acc + window * out
    out_ref[...] = jnp.tanh(x) * acc


@jax.jit
def _fbpinn(x, W_in, b_in, W_hid, b_hid, W_out, b_out):
    f32 = jnp.float32
    x, W_in, b_in, W_hid, b_hid, W_out, b_out = (
        x.astype(f32), W_in.astype(f32), b_in.astype(f32), W_hid.astype(f32),
        b_hid.astype(f32), W_out.astype(f32), b_out.astype(f32))
    n = x.shape[0]
    grid = n // _TILE
    x2 = x.reshape(1, n)
    inv_std = (1.0 / _STD)[:, None]
    w1 = (W_in[:, 0, :] * inv_std).T
    b1 = (b_in - (_MEANS[:, None] * inv_std) * W_in[:, 0, :]).T
    w_hid2 = W_hid.reshape(_NW * (_HIDDEN - 1) * _NEURONS, _NEURONS)
    b_hid_t = b_hid.transpose(2, 0, 1).reshape(_NEURONS, _NW * (_HIDDEN - 1))
    w_out_t = W_out[:, :, 0].T
    b_out_t = b_out.T
    full = lambda *s: pl.BlockSpec(s, lambda i: (0,) * len(s))
    pred = pl.pallas_call(
        _fbpinn_kernel,
        grid=(grid,),
        in_specs=[
            pl.BlockSpec((1, _TILE), lambda i: (0, i)),
            full(_NEURONS, _NW),
            full(_NEURONS, _NW),
            full(_NW * (_HIDDEN - 1) * _NEURONS, _NEURONS),
            full(_NEURONS, _NW * (_HIDDEN - 1)),
            full(_NEURONS, _NW),
            full(1, _NW),
        ],
        out_specs=pl.BlockSpec((1, _TILE), lambda i: (0, i)),
        out_shape=jax.ShapeDtypeStruct((1, n), jnp.float32),
    )(x2, w1, b1, w_hid2, b_hid_t, w_out_t, b_out_t)
    return pred.reshape(n)


def kernel(input, W_in, b_in, W_hid, b_hid, W_out, b_out):
    with jax.enable_x64(False):
        pred = _fbpinn(input, W_in, b_in, W_hid, b_hid, W_out, b_out)
    pred = pred.astype(jnp.promote_types(W_hid.dtype, jnp.float32))
    flops = np.int64(_NW * _N_PTS * (2 * _NEURONS +
                                     (_HIDDEN - 1) * 2 * _NEURONS * _NEURONS +
                                     2 * _NEURONS))
    return (pred, flops)


# speedup vs baseline: 411.8745x; 1.9785x over previous
import jax
import jax.numpy as jnp
import numpy as np
from jax.experimental import pallas as pl

jax.config.update("jax_enable_x64", True)

_NW = 16
_HIDDEN = 4
_NEURONS = 128
_OVERLAP = 0.25
_SIGMA = 0.02
_DOM = (0.0, 1.0)
_N_PTS = 131072

_TILE = 2048


def _geometry():
    w = (_DOM[1] - _DOM[0]) / _NW
    sd = np.zeros((_NW, 2), dtype=np.float32)
    for i in range(_NW):
        sd[i, 0] = _DOM[0] if i == 0 else _DOM[0] + (i - _OVERLAP / 2) * w
        sd[i, 1] = _DOM[1] if i == _NW - 1 else _DOM[0] + (i + 1 + _OVERLAP / 2) * w
    mo = np.zeros(_NW + 1, dtype=np.float32)
    mo[0] = sd[0, 0]
    mo[_NW] = sd[_NW - 1, 1]
    for i in range(1, _NW):
        mo[i] = (sd[i - 1, 1] + sd[i, 0]) / 2
    means = (sd[:, 1] + sd[:, 0]) / 2
    std = (sd[:, 1] - sd[:, 0]) / 2
    return mo, means, std


_MO, _MEANS, _STD = _geometry()


def _dot_t(w, h):
    return jax.lax.dot_general(w, h, (((0,), (0,)), ((), ())),
                               preferred_element_type=jnp.float32)


def _fbpinn_kernel(x_ref, w1_ref, b1_ref, wh_hi_ref, wh_lo_ref, b_hid_ref,
                   w_out_ref, b_out_ref, out_ref):
    bf16, f32 = jnp.bfloat16, jnp.float32
    x = x_ref[...]
    sig = [jax.nn.sigmoid((x - float(_MO[k])) * (1.0 / _SIGMA))
           for k in range(_NW + 1)]
    acc = jnp.zeros_like(x)
    for i in range(_NW):
        window = sig[i] * (1.0 - sig[i + 1])
        h = jnp.tanh(w1_ref[:, i:i + 1] * x + b1_ref[:, i:i + 1])
        for j in range(_HIDDEN - 1):
            base = (i * (_HIDDEN - 1) + j) * _NEURONS
            w_hi = wh_hi_ref[base:base + _NEURONS, :]
            w_lo = wh_lo_ref[base:base + _NEURONS, :]
            h_hi = h.astype(bf16)
            h_lo = (h - h_hi.astype(f32)).astype(bf16)
            z = (_dot_t(w_hi, h_hi) + _dot_t(w_hi, h_lo)
                 + _dot_t(w_lo, h_hi))
            h = jnp.tanh(z + b_hid_ref[:, 3 * i + j:3 * i + j + 1])
        out = (jnp.sum(h * w_out_ref[:, i:i + 1], axis=0, keepdims=True)
               + b_out_ref[:, i:i + 1])
        acc = acc + window * out
    out_ref[...] = jnp.tanh(x) * acc


@jax.jit
def _fbpinn(x, W_in, b_in, W_hid, b_hid, W_out, b_out):
    f32 = jnp.float32
    x, W_in, b_in, W_hid, b_hid, W_out, b_out = (
        x.astype(f32), W_in.astype(f32), b_in.astype(f32), W_hid.astype(f32),
        b_hid.astype(f32), W_out.astype(f32), b_out.astype(f32))
    n = x.shape[0]
    grid = n // _TILE
    x2 = x.reshape(1, n)
    inv_std = (1.0 / _STD)[:, None]
    w1 = (W_in[:, 0, :] * inv_std).T
    b1 = (b_in - (_MEANS[:, None] * inv_std) * W_in[:, 0, :]).T
    w_hid2 = W_hid.reshape(_NW * (_HIDDEN - 1) * _NEURONS, _NEURONS)
    wh_hi = w_hid2.astype(jnp.bfloat16)
    wh_lo = (w_hid2 - wh_hi.astype(f32)).astype(jnp.bfloat16)
    b_hid_t = b_hid.transpose(2, 0, 1).reshape(_NEURONS, _NW * (_HIDDEN - 1))
    w_out_t = W_out[:, :, 0].T
    b_out_t = b_out.T
    full = lambda *s: pl.BlockSpec(s, lambda i: (0,) * len(s))
    pred = pl.pallas_call(
        _fbpinn_kernel,
        grid=(grid,),
        in_specs=[
            pl.BlockSpec((1, _TILE), lambda i: (0, i)),
            full(_NEURONS, _NW),
            full(_NEURONS, _NW),
            full(_NW * (_HIDDEN - 1) * _NEURONS, _NEURONS),
            full(_NW * (_HIDDEN - 1) * _NEURONS, _NEURONS),
            full(_NEURONS, _NW * (_HIDDEN - 1)),
            full(_NEURONS, _NW),
            full(1, _NW),
        ],
        out_specs=pl.BlockSpec((1, _TILE), lambda i: (0, i)),
        out_shape=jax.ShapeDtypeStruct((1, n), jnp.float32),
    )(x2, w1, b1, wh_hi, wh_lo, b_hid_t, w_out_t, b_out_t)
    return pred.reshape(n)


def kernel(input, W_in, b_in, W_hid, b_hid, W_out, b_out):
    with jax.enable_x64(False):
        pred = _fbpinn(input, W_in, b_in, W_hid, b_hid, W_out, b_out)
    pred = pred.astype(jnp.promote_types(W_hid.dtype, jnp.float32))
    flops = np.int64(_NW * _N_PTS * (2 * _NEURONS +
                                     (_HIDDEN - 1) * 2 * _NEURONS * _NEURONS +
                                     2 * _NEURONS))
    return (pred, flops)


# speedup vs baseline: 416.4034x; 1.0110x over previous
import jax
import jax.numpy as jnp
import numpy as np
from jax.experimental import pallas as pl

jax.config.update("jax_enable_x64", True)

_NW = 16
_HIDDEN = 4
_NEURONS = 128
_OVERLAP = 0.25
_SIGMA = 0.02
_DOM = (0.0, 1.0)
_N_PTS = 131072

_TILE = 2048


def _geometry():
    w = (_DOM[1] - _DOM[0]) / _NW
    sd = np.zeros((_NW, 2), dtype=np.float32)
    for i in range(_NW):
        sd[i, 0] = _DOM[0] if i == 0 else _DOM[0] + (i - _OVERLAP / 2) * w
        sd[i, 1] = _DOM[1] if i == _NW - 1 else _DOM[0] + (i + 1 + _OVERLAP / 2) * w
    mo = np.zeros(_NW + 1, dtype=np.float32)
    mo[0] = sd[0, 0]
    mo[_NW] = sd[_NW - 1, 1]
    for i in range(1, _NW):
        mo[i] = (sd[i - 1, 1] + sd[i, 0]) / 2
    means = (sd[:, 1] + sd[:, 0]) / 2
    std = (sd[:, 1] - sd[:, 0]) / 2
    return mo, means, std


_MO, _MEANS, _STD = _geometry()


def _dot_t(w, h):
    return jax.lax.dot_general(w, h, (((0,), (0,)), ((), ())),
                               preferred_element_type=jnp.float32)


def _fbpinn_kernel(x_ref, w1_ref, b1_ref, wh_hi_ref, wh_lo_ref, b_hid_ref,
                   w_out_ref, b_out_ref, out_ref):
    bf16, f32 = jnp.bfloat16, jnp.float32
    x = x_ref[...]
    sig = [jax.nn.sigmoid((x - float(_MO[k])) * (1.0 / _SIGMA))
           for k in range(_NW + 1)]
    acc = jnp.zeros_like(x)
    for i in range(_NW):
        window = sig[i] * (1.0 - sig[i + 1])
        h = jnp.tanh(w1_ref[:, i:i + 1] * x + b1_ref[:, i:i + 1])
        for j in range(_HIDDEN - 1):
            base = (i * (_HIDDEN - 1) + j) * _NEURONS
            w_hi = wh_hi_ref[base:base + _NEURONS, :]
            w_lo = wh_lo_ref[base:base + _NEURONS, :]
            h_hi = h.astype(bf16)
            h_lo = (h - h_hi.astype(f32)).astype(bf16)
            z = (_dot_t(w_hi, h_hi) + _dot_t(w_hi, h_lo)
                 + _dot_t(w_lo, h_hi))
            h = jnp.tanh(z + b_hid_ref[:, 3 * i + j:3 * i + j + 1])
        out = (jnp.sum(h * w_out_ref[:, i:i + 1], axis=0, keepdims=True)
               + b_out_ref[:, i:i + 1])
        acc = acc + window * out
    out_ref[...] = jnp.tanh(x) * acc


@jax.jit
def _fbpinn(x, W_in, b_in, W_hid, b_hid, W_out, b_out):
    f32 = jnp.float32
    x, W_in, b_in, W_hid, b_hid, W_out, b_out = (
        x.astype(f32), W_in.astype(f32), b_in.astype(f32), W_hid.astype(f32),
        b_hid.astype(f32), W_out.astype(f32), b_out.astype(f32))
    n = x.shape[0]
    grid = n // _TILE
    x2 = x.reshape(1, n)
    inv_std = (1.0 / _STD)[:, None]
    w1 = (W_in[:, 0, :] * inv_std).T
    b1 = (b_in - (_MEANS[:, None] * inv_std) * W_in[:, 0, :]).T
    w_hid2 = W_hid.reshape(_NW * (_HIDDEN - 1) * _NEURONS, _NEURONS)
    bits = jax.lax.bitcast_convert_type(w_hid2, jnp.uint32)
    hi_f32 = jax.lax.bitcast_convert_type(
        bits & np.uint32(0xFFFF0000), jnp.float32)
    wh_hi = hi_f32.astype(jnp.bfloat16)
    wh_lo = (w_hid2 - hi_f32).astype(jnp.bfloat16)
    b_hid_t = b_hid.transpose(2, 0, 1).reshape(_NEURONS, _NW * (_HIDDEN - 1))
    w_out_t = W_out[:, :, 0].T
    b_out_t = b_out.T
    full = lambda *s: pl.BlockSpec(s, lambda i: (0,) * len(s))
    pred = pl.pallas_call(
        _fbpinn_kernel,
        grid=(grid,),
        in_specs=[
            pl.BlockSpec((1, _TILE), lambda i: (0, i)),
            full(_NEURONS, _NW),
            full(_NEURONS, _NW),
            full(_NW * (_HIDDEN - 1) * _NEURONS, _NEURONS),
            full(_NW * (_HIDDEN - 1) * _NEURONS, _NEURONS),
            full(_NEURONS, _NW * (_HIDDEN - 1)),
            full(_NEURONS, _NW),
            full(1, _NW),
        ],
        out_specs=pl.BlockSpec((1, _TILE), lambda i: (0, i)),
        out_shape=jax.ShapeDtypeStruct((1, n), jnp.float32),
    )(x2, w1, b1, wh_hi, wh_lo, b_hid_t, w_out_t, b_out_t)
    return pred.reshape(n)


def kernel(input, W_in, b_in, W_hid, b_hid, W_out, b_out):
    with jax.enable_x64(False):
        pred = _fbpinn(input, W_in, b_in, W_hid, b_hid, W_out, b_out)
    pred = pred.astype(jnp.promote_types(W_hid.dtype, jnp.float32))
    flops = np.int64(_NW * _N_PTS * (2 * _NEURONS +
                                     (_HIDDEN - 1) * 2 * _NEURONS * _NEURONS +
                                     2 * _NEURONS))
    return (pred, flops)
